# Initial kernel scaffold; baseline (speedup 1.0000x reference)
import jax
import jax.numpy as jnp
from jax.experimental import pallas as pl


def kernel(h_pair, pair_vu_idx, pair_uw_idx, pair_vw_idx, geom_features, psi_W1, psi_b1, psi_W2, psi_b2, phi_W1, phi_b1, phi_W2, phi_b2):
    raise NotImplementedError("write your pallas kernel here")


# speedup vs baseline: 1.1756x; 1.1756x over previous
import functools
import os

import jax
import jax.numpy as jnp
from jax import lax
from jax.experimental import pallas as pl
from jax.experimental.pallas import tpu as pltpu
from jax.experimental.pallas import tpu_sc as plsc

_D = 128


def _precompute_body(x_ref, w_ref, oa, ob, oc, od):
    y = jnp.dot(x_ref[...], w_ref[...], preferred_element_type=jnp.float32)
    oa[...] = y[:, 0 * _D:1 * _D]
    ob[...] = y[:, 1 * _D:2 * _D]
    oc[...] = y[:, 2 * _D:3 * _D]
    od[...] = y[:, 3 * _D:4 * _D]


def _precompute(h, wcat):
    P = h.shape[0]
    BP = 1600
    o = jax.ShapeDtypeStruct((P, _D), jnp.float32)
    return pl.pallas_call(
        _precompute_body,
        grid=(P // BP,),
        in_specs=[pl.BlockSpec((BP, _D), lambda i: (i, 0)),
                  pl.BlockSpec((_D, 4 * _D), lambda i: (0, 0))],
        out_specs=[pl.BlockSpec((BP, _D), lambda i: (i, 0))] * 4,
        out_shape=[o, o, o, o],
    )(h, wcat)


def _psi_tail_body(g_ref, geom_ref, gw_ref, b1_ref, w2_ref, b2_ref, o_ref):
    h1 = (g_ref[...]
          + jnp.dot(geom_ref[...], gw_ref[...], preferred_element_type=jnp.float32)
          + b1_ref[...])
    y = h1 * jax.nn.sigmoid(h1)
    o_ref[...] = jnp.dot(y, w2_ref[...], preferred_element_type=jnp.float32) + b2_ref[...]


def _psi_tail(g, geom, gw, b1, w2, b2):
    T = g.shape[0]
    BT = 2560
    G = geom.shape[1]
    return pl.pallas_call(
        _psi_tail_body,
        grid=(T // BT,),
        in_specs=[pl.BlockSpec((BT, _D), lambda i: (i, 0)),
                  pl.BlockSpec((BT, G), lambda i: (i, 0)),
                  pl.BlockSpec((G, _D), lambda i: (0, 0)),
                  pl.BlockSpec((1, _D), lambda i: (0, 0)),
                  pl.BlockSpec((_D, _D), lambda i: (0, 0)),
                  pl.BlockSpec((1, _D), lambda i: (0, 0))],
        out_specs=pl.BlockSpec((BT, _D), lambda i: (i, 0)),
        out_shape=jax.ShapeDtypeStruct((T, _D), jnp.float32),
    )(g, geom, gw, b1, w2, b2)


def _phi_body(h_ref, hd_ref, agg_ref, wb_ref, b1_ref, w2_ref, b2_ref, o_ref):
    z = (hd_ref[...]
         + jnp.dot(agg_ref[...], wb_ref[...], preferred_element_type=jnp.float32)
         + b1_ref[...])
    y = z * jax.nn.sigmoid(z)
    o_ref[...] = h_ref[...] + jnp.dot(y, w2_ref[...], preferred_element_type=jnp.float32) + b2_ref[...]


def _phi(h, hd, agg, wb, b1, w2, b2):
    P = h.shape[0]
    BP = 1600
    return pl.pallas_call(
        _phi_body,
        grid=(P // BP,),
        in_specs=[pl.BlockSpec((BP, _D), lambda i: (i, 0)),
                  pl.BlockSpec((BP, _D), lambda i: (i, 0)),
                  pl.BlockSpec((BP, _D), lambda i: (i, 0)),
                  pl.BlockSpec((_D, _D), lambda i: (0, 0)),
                  pl.BlockSpec((1, _D), lambda i: (0, 0)),
                  pl.BlockSpec((_D, _D), lambda i: (0, 0)),
                  pl.BlockSpec((1, _D), lambda i: (0, 0))],
        out_specs=pl.BlockSpec((BP, _D), lambda i: (i, 0)),
        out_shape=jax.ShapeDtypeStruct((P, _D), jnp.float32),
    )(h, hd, agg, wb, b1, w2, b2)


_NC = 2
_NS = 16


def _gather_sum(ha, hb, hc, vu, uw, vw):
    T = vu.shape[0]
    NW = _NC * _NS
    CH = T // NW
    W = 80
    NWIN = CH // W
    mesh = plsc.VectorSubcoreMesh(core_axis_name="c", subcore_axis_name="s")

    @functools.partial(
        pl.kernel, mesh=mesh,
        out_type=jax.ShapeDtypeStruct((T, _D), jnp.float32),
        scratch_types=[
            pltpu.VMEM((CH,), jnp.int32),
            pltpu.VMEM((CH,), jnp.int32),
            pltpu.VMEM((CH,), jnp.int32),
            pltpu.VMEM((W, _D), jnp.float32),
            pltpu.VMEM((W, _D), jnp.float32),
            pltpu.VMEM((W, _D), jnp.float32),
            pltpu.VMEM((W, _D), jnp.float32),
            pltpu.SemaphoreType.DMA,
        ],
    )
    def k(ha_h, hb_h, hc_h, vu_h, uw_h, vw_h, out_h,
          vub, uwb, vwb, ra, rb, rc, ob, sem):
        wid = lax.axis_index("s") * _NC + lax.axis_index("c")
        base = wid * CH
        pltpu.sync_copy(vu_h.at[pl.ds(base, CH)], vub)
        pltpu.sync_copy(uw_h.at[pl.ds(base, CH)], uwb)
        pltpu.sync_copy(vw_h.at[pl.ds(base, CH)], vwb)

        def win(w, _):
            wb = w * W
            c1 = pltpu.async_copy(ha_h.at[vub.at[pl.ds(wb, W)]], ra, sem)
            c2 = pltpu.async_copy(hb_h.at[uwb.at[pl.ds(wb, W)]], rb, sem)
            c3 = pltpu.async_copy(hc_h.at[vwb.at[pl.ds(wb, W)]], rc, sem)
            c1.wait()
            c2.wait()
            c3.wait()

            def row(r, _):
                for c in range(_D // 16):
                    s = pl.ds(c * 16, 16)
                    ob[r, s] = ra[r, s] + rb[r, s] + rc[r, s]
                return 0

            lax.fori_loop(0, W, row, 0)
            pltpu.sync_copy(ob, out_h.at[pl.ds(base + wb, W)])
            return 0

        lax.fori_loop(0, NWIN, win, 0)

    return k(ha, hb, hc, vu, uw, vw)


def _scatter_add(m, vw, P):
    T = m.shape[0]
    R = 8000
    RP = 8192
    NPASS = P // (_NC * R)
    W = 32
    CH = T // _NS
    NG = CH // 16
    ZT = RP // _NS
    ZR = 32
    OPT = 496
    mesh = plsc.VectorSubcoreMesh(core_axis_name="c", subcore_axis_name="s")

    @functools.partial(
        pl.kernel, mesh=mesh,
        out_type=jax.ShapeDtypeStruct((P, _D), jnp.float32),
        scratch_types=[
            pltpu.VMEM((CH,), jnp.int32),
            pltpu.VMEM((CH + 32,), jnp.int32),
            pltpu.VMEM((CH + 32,), jnp.int32),
            pltpu.VMEM((W,), jnp.int32),
            pltpu.VMEM((32,), jnp.int32),
            pltpu.VMEM((W, _D), jnp.float32),
            pltpu.VMEM_SHARED((RP, _D), jnp.float32),
            pltpu.SemaphoreType.DMA,
        ],
    )
    def k(m_h, vw_h, agg_h, vwb, glist, rlist, dwin, sbuf, mbuf, sdest, sem1):
        cid = lax.axis_index("c")
        sid = lax.axis_index("s")
        base_t = pl.multiple_of(sid * CH, 8)
        pltpu.sync_copy(vw_h.at[pl.ds(base_t, CH)], vwb)

        z16f = jnp.zeros((16,), jnp.float32)
        z16i = jnp.zeros((16,), jnp.int32)
        pad16 = jnp.full((16,), R, jnp.int32)
        sbuf[pl.ds(0, 16)] = z16i
        sbuf[pl.ds(16, 16)] = z16i

        for p in range(NPASS):
            lo = p * _NC * R + cid * R

            def zmb(r, _):
                for c in range(_D // 16):
                    mbuf[r, pl.ds(c * 16, 16)] = z16f
                return 0
            lax.fori_loop(0, ZR, zmb, 0)
            for z in range(ZT // ZR):
                zoff = pl.multiple_of(sid * ZT + z * ZR, 8)
                pltpu.sync_copy(mbuf.at[pl.ds(0, ZR)], sdest.at[pl.ds(zoff, ZR)])
            plsc.subcore_barrier()

            def flt(g, cnt):
                v = vwb[pl.ds(g * 16, 16)]
                rel = v - lo
                t0 = rel | ((R - 1) - rel)
                mi = 1 + (t0 >> 31)
                garb = R + (g & 127)
                dest = rel * mi + garb * (1 - mi)
                s = mi
                for kk in (1, 2, 4, 8):
                    sbuf[pl.ds(0, 16)] = s
                    s = s + sbuf[pl.ds(kk, 16)]
                total = s[0]
                any_hit = (total + 15) >> 4

                @pl.when(total > 0)
                def _append():
                    rlist[pl.ds(cnt, 16)] = dest
                    glist[pl.ds(cnt, 16)] = z16i + (base_t + g * 16)
                return cnt + any_hit * 16
            cnt = lax.fori_loop(0, NG, flt, jnp.int32(0))

            rlist[pl.ds(cnt, 16)] = pad16
            glist[pl.ds(cnt, 16)] = z16i + base_t

            nw = (cnt + (W - 1)) >> 5

            def drain(w, _):
                wb = pl.multiple_of(w * W, 8)
                gv1 = glist[pl.ds(wb, 16)]
                gv2 = glist[pl.ds(wb + 16, 16)]
                t1 = pl.multiple_of(gv1[0], 8)
                t2 = pl.multiple_of(gv2[0], 8)
                c1 = pltpu.async_copy(m_h.at[pl.ds(t1, 16)],
                                      mbuf.at[pl.ds(0, 16)], sem1)
                c2 = pltpu.async_copy(m_h.at[pl.ds(t2, 16)],
                                      mbuf.at[pl.ds(16, 16)], sem1)
                dwin[pl.ds(0, 16)] = rlist[pl.ds(wb, 16)]
                dwin[pl.ds(16, 16)] = rlist[pl.ds(wb + 16, 16)]
                c1.wait()
                c2.wait()
                pltpu.sync_copy(mbuf, sdest.at[dwin], add=True)
                return 0
            lax.fori_loop(0, nw, drain, 0)
            plsc.subcore_barrier()

            ooff = pl.multiple_of(sid * OPT, 8)
            pltpu.sync_copy(sdest.at[pl.ds(ooff, OPT)],
                            agg_h.at[pl.ds(pl.multiple_of(lo, 8) + ooff, OPT)])

            @pl.when(sid == _NS - 1)
            def _tail():
                pltpu.sync_copy(
                    sdest.at[pl.ds(_NS * OPT, R - _NS * OPT)],
                    agg_h.at[pl.ds(pl.multiple_of(lo, 8) + _NS * OPT,
                                   R - _NS * OPT)])
            plsc.subcore_barrier()

    return k(m, vw)


def kernel(h_pair, pair_vu_idx, pair_uw_idx, pair_vw_idx, geom_features,
           psi_W1, psi_b1, psi_W2, psi_b2, phi_W1, phi_b1, phi_W2, phi_b2):
    P, D = h_pair.shape
    vu = pair_vu_idx.astype(jnp.int32)
    uw = pair_uw_idx.astype(jnp.int32)
    vw = pair_vw_idx.astype(jnp.int32)

    wcat = jnp.concatenate(
        [psi_W1[0:D], psi_W1[D:2 * D], psi_W1[2 * D:3 * D], phi_W1[0:D]], axis=1)
    ha, hb, hc, hd = _precompute(h_pair, wcat)
    g = _gather_sum(ha, hb, hc, vu, uw, vw)
    m = _psi_tail(g, geom_features, psi_W1[3 * D:],
                  psi_b1.reshape(1, D), psi_W2, psi_b2.reshape(1, D))
    agg = _scatter_add(m, vw, P)
    out = _phi(h_pair, hd, agg, phi_W1[D:], phi_b1.reshape(1, D),
               phi_W2, phi_b2.reshape(1, D))
    return out


# speedup vs baseline: 1.8158x; 1.5445x over previous
import functools
import os

import jax
import jax.numpy as jnp
from jax import lax
from jax.experimental import pallas as pl
from jax.experimental.pallas import tpu as pltpu
from jax.experimental.pallas import tpu_sc as plsc

_D = 128


def _precompute_body(x_ref, w_ref, oa, ob, oc, od):
    y = jnp.dot(x_ref[...], w_ref[...], preferred_element_type=jnp.float32)
    oa[...] = y[:, 0 * _D:1 * _D]
    ob[...] = y[:, 1 * _D:2 * _D]
    oc[...] = y[:, 2 * _D:3 * _D]
    od[...] = y[:, 3 * _D:4 * _D]


def _precompute(h, wcat):
    P = h.shape[0]
    BP = 1600
    o = jax.ShapeDtypeStruct((P, _D), jnp.float32)
    return pl.pallas_call(
        _precompute_body,
        grid=(P // BP,),
        in_specs=[pl.BlockSpec((BP, _D), lambda i: (i, 0)),
                  pl.BlockSpec((_D, 4 * _D), lambda i: (0, 0))],
        out_specs=[pl.BlockSpec((BP, _D), lambda i: (i, 0))] * 4,
        out_shape=[o, o, o, o],
    )(h, wcat)


def _psi_tail_body(g_ref, geom_ref, gw_ref, b1_ref, w2_ref, b2_ref, o_ref):
    h1 = (g_ref[...]
          + jnp.dot(geom_ref[...], gw_ref[...], preferred_element_type=jnp.float32)
          + b1_ref[...])
    y = h1 * jax.nn.sigmoid(h1)
    o_ref[...] = jnp.dot(y, w2_ref[...], preferred_element_type=jnp.float32) + b2_ref[...]


def _psi_tail(g, geom, gw, b1, w2, b2):
    T = g.shape[0]
    BT = 2560
    G = geom.shape[1]
    return pl.pallas_call(
        _psi_tail_body,
        grid=(T // BT,),
        in_specs=[pl.BlockSpec((BT, _D), lambda i: (i, 0)),
                  pl.BlockSpec((BT, G), lambda i: (i, 0)),
                  pl.BlockSpec((G, _D), lambda i: (0, 0)),
                  pl.BlockSpec((1, _D), lambda i: (0, 0)),
                  pl.BlockSpec((_D, _D), lambda i: (0, 0)),
                  pl.BlockSpec((1, _D), lambda i: (0, 0))],
        out_specs=pl.BlockSpec((BT, _D), lambda i: (i, 0)),
        out_shape=jax.ShapeDtypeStruct((T, _D), jnp.float32),
    )(g, geom, gw, b1, w2, b2)


def _phi_body(h_ref, hd_ref, agg_ref, wb_ref, b1_ref, w2_ref, b2_ref, o_ref):
    z = (hd_ref[...]
         + jnp.dot(agg_ref[...], wb_ref[...], preferred_element_type=jnp.float32)
         + b1_ref[...])
    y = z * jax.nn.sigmoid(z)
    o_ref[...] = h_ref[...] + jnp.dot(y, w2_ref[...], preferred_element_type=jnp.float32) + b2_ref[...]


def _phi(h, hd, agg, wb, b1, w2, b2):
    P = h.shape[0]
    BP = 1600
    return pl.pallas_call(
        _phi_body,
        grid=(P // BP,),
        in_specs=[pl.BlockSpec((BP, _D), lambda i: (i, 0)),
                  pl.BlockSpec((BP, _D), lambda i: (i, 0)),
                  pl.BlockSpec((BP, _D), lambda i: (i, 0)),
                  pl.BlockSpec((_D, _D), lambda i: (0, 0)),
                  pl.BlockSpec((1, _D), lambda i: (0, 0)),
                  pl.BlockSpec((_D, _D), lambda i: (0, 0)),
                  pl.BlockSpec((1, _D), lambda i: (0, 0))],
        out_specs=pl.BlockSpec((BP, _D), lambda i: (i, 0)),
        out_shape=jax.ShapeDtypeStruct((P, _D), jnp.float32),
    )(h, hd, agg, wb, b1, w2, b2)


_NC = 2
_NS = 16


def _gather_sum(ha, hb, hc, vu, uw, vw):
    T = vu.shape[0]
    NW = _NC * _NS
    CH = T // NW
    W = 80
    NWIN = CH // W
    mesh = plsc.VectorSubcoreMesh(core_axis_name="c", subcore_axis_name="s")

    @functools.partial(
        pl.kernel, mesh=mesh,
        out_type=jax.ShapeDtypeStruct((T, _D), jnp.float32),
        scratch_types=[
            pltpu.VMEM((CH,), jnp.int32),
            pltpu.VMEM((CH,), jnp.int32),
            pltpu.VMEM((CH,), jnp.int32),
            pltpu.VMEM((W, _D), jnp.float32),
            pltpu.VMEM((W, _D), jnp.float32),
            pltpu.VMEM((W, _D), jnp.float32),
            pltpu.VMEM((W, _D), jnp.float32),
            pltpu.SemaphoreType.DMA,
        ],
    )
    def k(ha_h, hb_h, hc_h, vu_h, uw_h, vw_h, out_h,
          vub, uwb, vwb, ra, rb, rc, ob, sem):
        wid = lax.axis_index("s") * _NC + lax.axis_index("c")
        base = wid * CH
        pltpu.sync_copy(vu_h.at[pl.ds(base, CH)], vub)
        pltpu.sync_copy(uw_h.at[pl.ds(base, CH)], uwb)
        pltpu.sync_copy(vw_h.at[pl.ds(base, CH)], vwb)

        def win(w, _):
            wb = w * W
            c1 = pltpu.async_copy(ha_h.at[vub.at[pl.ds(wb, W)]], ra, sem)
            c2 = pltpu.async_copy(hb_h.at[uwb.at[pl.ds(wb, W)]], rb, sem)
            c3 = pltpu.async_copy(hc_h.at[vwb.at[pl.ds(wb, W)]], rc, sem)
            c1.wait()
            c2.wait()
            c3.wait()

            def row(r, _):
                for c in range(_D // 16):
                    s = pl.ds(c * 16, 16)
                    ob[r, s] = ra[r, s] + rb[r, s] + rc[r, s]
                return 0

            lax.fori_loop(0, W, row, 0)
            pltpu.sync_copy(ob, out_h.at[pl.ds(base + wb, W)])
            return 0

        lax.fori_loop(0, NWIN, win, 0)

    return k(ha, hb, hc, vu, uw, vw)


def _scatter_add(m, vw, P):
    T = m.shape[0]
    R = 8000
    RP = 8192
    NPASS = P // (_NC * R)
    W = 128
    GPW = W // 16
    CH = T // _NS
    NG = CH // 16
    ZT = RP // _NS
    ZR = 128
    OPT = 496
    mesh = plsc.VectorSubcoreMesh(core_axis_name="c", subcore_axis_name="s")

    @functools.partial(
        pl.kernel, mesh=mesh,
        out_type=jax.ShapeDtypeStruct((P, _D), jnp.float32),
        scratch_types=[
            pltpu.VMEM((CH,), jnp.int32),
            pltpu.VMEM((NG + 32,), jnp.int32),
            pltpu.VMEM((CH + 16 * GPW,), jnp.int32),
            pltpu.VMEM((W,), jnp.int32),
            pltpu.VMEM((32,), jnp.int32),
            pltpu.VMEM((W, _D), jnp.float32),
            pltpu.VMEM_SHARED((RP, _D), jnp.float32),
            pltpu.SemaphoreType.DMA,
        ],
    )
    def k(m_h, vw_h, agg_h, vwb, glist, rlist, dwin, sbuf, mbuf, sdest, sem1):
        cid = lax.axis_index("c")
        sid = lax.axis_index("s")
        base_t = pl.multiple_of(sid * CH, 8)
        pltpu.sync_copy(vw_h.at[pl.ds(base_t, CH)], vwb)

        z16f = jnp.zeros((16,), jnp.float32)
        z16i = jnp.zeros((16,), jnp.int32)
        pad16 = jnp.full((16,), R, jnp.int32)
        sbuf[pl.ds(0, 16)] = z16i
        sbuf[pl.ds(16, 16)] = z16i

        for p in range(NPASS):
            lo = p * _NC * R + cid * R

            def zmb(r, _):
                for c in range(_D // 16):
                    mbuf[r, pl.ds(c * 16, 16)] = z16f
                return 0
            lax.fori_loop(0, ZR, zmb, 0)
            for z in range(ZT // ZR):
                zoff = pl.multiple_of(sid * ZT + z * ZR, 8)
                pltpu.sync_copy(mbuf.at[pl.ds(0, ZR)], sdest.at[pl.ds(zoff, ZR)])
            plsc.subcore_barrier()

            def flt(g, gcnt):
                v = vwb[pl.ds(g * 16, 16)]
                rel = v - lo
                t0 = rel | ((R - 1) - rel)
                mi = 1 + (t0 >> 31)
                garb = R + (g & 127)
                dest = rel * mi + garb * (1 - mi)
                s = mi
                for kk in (1, 2, 4, 8):
                    sbuf[pl.ds(0, 16)] = s
                    s = s + sbuf[pl.ds(kk, 16)]
                total = s[0]
                any_hit = (total + 15) >> 4

                @pl.when(total > 0)
                def _append():
                    rlist[pl.ds(gcnt * 16, 16)] = dest
                    glist[pl.ds(gcnt, 16)] = z16i + (base_t + g * 16)
                return gcnt + any_hit
            gcnt = lax.fori_loop(0, NG, flt, jnp.int32(0))

            glist[pl.ds(gcnt, 16)] = z16i + base_t
            for q in range(GPW):
                rlist[pl.ds((gcnt + q) * 16, 16)] = pad16

            nw = (gcnt + (GPW - 1)) >> 3

            def drain(w, _):
                gb = w * GPW
                cps = []
                for q in range(GPW):
                    gv = glist[pl.ds(gb + q, 16)]
                    tq = pl.multiple_of(gv[0], 8)
                    cps.append(pltpu.async_copy(
                        m_h.at[pl.ds(tq, 16)],
                        mbuf.at[pl.ds(q * 16, 16)], sem1))
                for q in range(GPW):
                    dwin[pl.ds(q * 16, 16)] = rlist[pl.ds((gb + q) * 16, 16)]
                for cp in cps:
                    cp.wait()
                pltpu.sync_copy(mbuf, sdest.at[dwin], add=True)
                return 0
            lax.fori_loop(0, nw, drain, 0)
            plsc.subcore_barrier()

            ooff = pl.multiple_of(sid * OPT, 8)
            pltpu.sync_copy(sdest.at[pl.ds(ooff, OPT)],
                            agg_h.at[pl.ds(pl.multiple_of(lo, 8) + ooff, OPT)])

            @pl.when(sid == _NS - 1)
            def _tail():
                pltpu.sync_copy(
                    sdest.at[pl.ds(_NS * OPT, R - _NS * OPT)],
                    agg_h.at[pl.ds(pl.multiple_of(lo, 8) + _NS * OPT,
                                   R - _NS * OPT)])
            plsc.subcore_barrier()

    return k(m, vw)


def kernel(h_pair, pair_vu_idx, pair_uw_idx, pair_vw_idx, geom_features,
           psi_W1, psi_b1, psi_W2, psi_b2, phi_W1, phi_b1, phi_W2, phi_b2):
    P, D = h_pair.shape
    vu = pair_vu_idx.astype(jnp.int32)
    uw = pair_uw_idx.astype(jnp.int32)
    vw = pair_vw_idx.astype(jnp.int32)

    wcat = jnp.concatenate(
        [psi_W1[0:D], psi_W1[D:2 * D], psi_W1[2 * D:3 * D], phi_W1[0:D]], axis=1)
    ha, hb, hc, hd = _precompute(h_pair, wcat)
    g = _gather_sum(ha, hb, hc, vu, uw, vw)
    m = _psi_tail(g, geom_features, psi_W1[3 * D:],
                  psi_b1.reshape(1, D), psi_W2, psi_b2.reshape(1, D))
    agg = _scatter_add(m, vw, P)
    out = _phi(h_pair, hd, agg, phi_W1[D:], phi_b1.reshape(1, D),
               phi_W2, phi_b2.reshape(1, D))
    return out


# speedup vs baseline: 1.8783x; 1.0345x over previous
import functools
import os

import jax
import jax.numpy as jnp
from jax import lax
from jax.experimental import pallas as pl
from jax.experimental.pallas import tpu as pltpu
from jax.experimental.pallas import tpu_sc as plsc

_D = 128


def _precompute_body(x_ref, w_ref, oa, ob, oc, od):
    y = jnp.dot(x_ref[...], w_ref[...], preferred_element_type=jnp.float32)
    oa[...] = y[:, 0 * _D:1 * _D]
    ob[...] = y[:, 1 * _D:2 * _D]
    oc[...] = y[:, 2 * _D:3 * _D]
    od[...] = y[:, 3 * _D:4 * _D]


def _precompute(h, wcat):
    P = h.shape[0]
    BP = 1600
    o = jax.ShapeDtypeStruct((P, _D), jnp.float32)
    return pl.pallas_call(
        _precompute_body,
        grid=(P // BP,),
        in_specs=[pl.BlockSpec((BP, _D), lambda i: (i, 0)),
                  pl.BlockSpec((_D, 4 * _D), lambda i: (0, 0))],
        out_specs=[pl.BlockSpec((BP, _D), lambda i: (i, 0))] * 4,
        out_shape=[o, o, o, o],
    )(h, wcat)


def _psi_tail_body(g_ref, geom_ref, gw_ref, b1_ref, w2_ref, b2_ref, o_ref):
    h1 = (g_ref[...]
          + jnp.dot(geom_ref[...], gw_ref[...], preferred_element_type=jnp.float32)
          + b1_ref[...])
    y = h1 * jax.nn.sigmoid(h1)
    o_ref[...] = jnp.dot(y, w2_ref[...], preferred_element_type=jnp.float32) + b2_ref[...]


def _psi_tail(g, geom, gw, b1, w2, b2):
    T = g.shape[0]
    BT = 2560
    G = geom.shape[1]
    return pl.pallas_call(
        _psi_tail_body,
        grid=(T // BT,),
        in_specs=[pl.BlockSpec((BT, _D), lambda i: (i, 0)),
                  pl.BlockSpec((BT, G), lambda i: (i, 0)),
                  pl.BlockSpec((G, _D), lambda i: (0, 0)),
                  pl.BlockSpec((1, _D), lambda i: (0, 0)),
                  pl.BlockSpec((_D, _D), lambda i: (0, 0)),
                  pl.BlockSpec((1, _D), lambda i: (0, 0))],
        out_specs=pl.BlockSpec((BT, _D), lambda i: (i, 0)),
        out_shape=jax.ShapeDtypeStruct((T, _D), jnp.float32),
    )(g, geom, gw, b1, w2, b2)


def _phi_body(h_ref, hd_ref, agg_ref, wb_ref, b1_ref, w2_ref, b2_ref, o_ref):
    z = (hd_ref[...]
         + jnp.dot(agg_ref[...], wb_ref[...], preferred_element_type=jnp.float32)
         + b1_ref[...])
    y = z * jax.nn.sigmoid(z)
    o_ref[...] = h_ref[...] + jnp.dot(y, w2_ref[...], preferred_element_type=jnp.float32) + b2_ref[...]


def _phi(h, hd, agg, wb, b1, w2, b2):
    P = h.shape[0]
    BP = 1600
    return pl.pallas_call(
        _phi_body,
        grid=(P // BP,),
        in_specs=[pl.BlockSpec((BP, _D), lambda i: (i, 0)),
                  pl.BlockSpec((BP, _D), lambda i: (i, 0)),
                  pl.BlockSpec((BP, _D), lambda i: (i, 0)),
                  pl.BlockSpec((_D, _D), lambda i: (0, 0)),
                  pl.BlockSpec((1, _D), lambda i: (0, 0)),
                  pl.BlockSpec((_D, _D), lambda i: (0, 0)),
                  pl.BlockSpec((1, _D), lambda i: (0, 0))],
        out_specs=pl.BlockSpec((BP, _D), lambda i: (i, 0)),
        out_shape=jax.ShapeDtypeStruct((P, _D), jnp.float32),
    )(h, hd, agg, wb, b1, w2, b2)


_NC = 2
_NS = 16


def _gather_sum(ha, hb, hc, vu, uw, vw):
    T = vu.shape[0]
    NW = _NC * _NS
    CH = T // NW
    W = 80
    NWIN = CH // W
    mesh = plsc.VectorSubcoreMesh(core_axis_name="c", subcore_axis_name="s")

    @functools.partial(
        pl.kernel, mesh=mesh,
        out_type=jax.ShapeDtypeStruct((T, _D), jnp.float32),
        scratch_types=[
            pltpu.VMEM((CH,), jnp.int32),
            pltpu.VMEM((CH,), jnp.int32),
            pltpu.VMEM((CH,), jnp.int32),
            pltpu.VMEM((W, _D), jnp.float32),
            pltpu.VMEM((W, _D), jnp.float32),
            pltpu.VMEM((W, _D), jnp.float32),
            pltpu.VMEM((W, _D), jnp.float32),
            pltpu.SemaphoreType.DMA,
        ],
    )
    def k(ha_h, hb_h, hc_h, vu_h, uw_h, vw_h, out_h,
          vub, uwb, vwb, ra, rb, rc, ob, sem):
        wid = lax.axis_index("s") * _NC + lax.axis_index("c")
        base = wid * CH
        pltpu.sync_copy(vu_h.at[pl.ds(base, CH)], vub)
        pltpu.sync_copy(uw_h.at[pl.ds(base, CH)], uwb)
        pltpu.sync_copy(vw_h.at[pl.ds(base, CH)], vwb)

        def win(w, _):
            wb = w * W
            c1 = pltpu.async_copy(ha_h.at[vub.at[pl.ds(wb, W)]], ra, sem)
            c2 = pltpu.async_copy(hb_h.at[uwb.at[pl.ds(wb, W)]], rb, sem)
            c3 = pltpu.async_copy(hc_h.at[vwb.at[pl.ds(wb, W)]], rc, sem)
            c1.wait()
            c2.wait()
            c3.wait()

            def row(r, _):
                for c in range(_D // 16):
                    s = pl.ds(c * 16, 16)
                    ob[r, s] = ra[r, s] + rb[r, s] + rc[r, s]
                return 0

            lax.fori_loop(0, W, row, 0)
            pltpu.sync_copy(ob, out_h.at[pl.ds(base + wb, W)])
            return 0

        lax.fori_loop(0, NWIN, win, 0)

    return k(ha, hb, hc, vu, uw, vw)


def _scatter_add(m, vw, P):
    T = m.shape[0]
    R = 8000
    RP = 8192
    NPASS = P // (_NC * R)
    W = 64
    GPW = W // 16
    CH = T // _NS
    NG = CH // 16
    ZT = RP // _NS
    ZR = 64
    OPT = 496
    mesh = plsc.VectorSubcoreMesh(core_axis_name="c", subcore_axis_name="s")

    @functools.partial(
        pl.kernel, mesh=mesh,
        out_type=jax.ShapeDtypeStruct((P, _D), jnp.float32),
        scratch_types=[
            pltpu.VMEM((CH,), jnp.int32),
            pltpu.VMEM((NG + 32,), jnp.int32),
            pltpu.VMEM((CH + 224,), jnp.int32),
            pltpu.VMEM((2, W), jnp.int32),
            pltpu.VMEM((32,), jnp.int32),
            pltpu.VMEM((W, _D), jnp.float32),
            pltpu.VMEM((W, _D), jnp.float32),
            pltpu.VMEM_SHARED((RP, _D), jnp.float32),
            pltpu.SemaphoreType.DMA,
            pltpu.SemaphoreType.DMA,
            pltpu.SemaphoreType.DMA,
            pltpu.SemaphoreType.DMA,
        ],
    )
    def k(m_h, vw_h, agg_h, vwb, glist, rlist, dwin, sbuf, mbuf, mbuf2,
          sdest, g0sem, g1sem, s0sem, s1sem):
        cid = lax.axis_index("c")
        sid = lax.axis_index("s")
        base_t = pl.multiple_of(sid * CH, 8)
        pltpu.sync_copy(vw_h.at[pl.ds(base_t, CH)], vwb)

        z16f = jnp.zeros((16,), jnp.float32)
        z16i = jnp.zeros((16,), jnp.int32)
        pad16 = jnp.full((16,), R, jnp.int32)
        sbuf[pl.ds(0, 16)] = z16i
        sbuf[pl.ds(16, 16)] = z16i

        for p in range(NPASS):
            lo = p * _NC * R + cid * R

            def zmb(r, _):
                for c in range(_D // 16):
                    mbuf[r, pl.ds(c * 16, 16)] = z16f
                return 0
            lax.fori_loop(0, ZR, zmb, 0)
            for z in range(ZT // ZR):
                zoff = pl.multiple_of(sid * ZT + z * ZR, 8)
                pltpu.sync_copy(mbuf.at[pl.ds(0, ZR)], sdest.at[pl.ds(zoff, ZR)])
            plsc.subcore_barrier()

            def flt(g, gcnt):
                v = vwb[pl.ds(g * 16, 16)]
                rel = v - lo
                t0 = rel | ((R - 1) - rel)
                mi = 1 + (t0 >> 31)
                garb = R + (g & 127)
                dest = rel * mi + garb * (1 - mi)
                s = mi
                for kk in (1, 2, 4, 8):
                    sbuf[pl.ds(0, 16)] = s
                    s = s + sbuf[pl.ds(kk, 16)]
                total = s[0]
                any_hit = (total + 15) >> 4

                @pl.when(total > 0)
                def _append():
                    rlist[pl.ds(gcnt * 16, 16)] = dest
                    glist[pl.ds(gcnt, 16)] = z16i + (base_t + g * 16)
                return gcnt + any_hit
            gcnt = lax.fori_loop(0, NG, flt, jnp.int32(0))

            glist[pl.ds(gcnt, 16)] = z16i + base_t
            for q in range(3 * GPW):
                rlist[pl.ds((gcnt + q) * 16, 16)] = pad16

            nw = (gcnt + (GPW - 1)) >> 2
            npairs = (nw + 1) >> 1

            def fire_gathers(w, buf, sem):
                gb = w * GPW
                for q in range(GPW):
                    gv = glist[pl.ds(gb + q, 16)]
                    tq = pl.multiple_of(gv[0], 8)
                    pltpu.async_copy(m_h.at[pl.ds(tq, 16)],
                                     buf.at[pl.ds(q * 16, 16)], sem)

            def build_dwin(w, slot):
                gb = w * GPW
                for q in range(GPW):
                    dwin[slot, pl.ds(q * 16, 16)] = rlist[pl.ds((gb + q) * 16, 16)]

            def dummy_wait(sem):
                pltpu.make_async_copy(m_h.at[pl.ds(0, W)], mbuf, sem).wait()

            fire_gathers(0, mbuf, g0sem)

            def pair(i, _):
                w0 = i * 2
                w1 = w0 + 1

                @pl.when(i >= 1)
                def _ws1():
                    dummy_wait(s1sem)
                fire_gathers(w1, mbuf2, g1sem)
                dummy_wait(g0sem)
                build_dwin(w0, 0)
                pltpu.async_copy(mbuf, sdest.at[dwin.at[0]], s0sem, add=True)
                dummy_wait(g1sem)
                build_dwin(w1, 1)
                pltpu.async_copy(mbuf2, sdest.at[dwin.at[1]], s1sem, add=True)
                dummy_wait(s0sem)
                fire_gathers(w0 + 2, mbuf, g0sem)
                return 0
            lax.fori_loop(0, npairs, pair, 0)

            @pl.when(npairs > 0)
            def _tail_s1():
                dummy_wait(s1sem)
            dummy_wait(g0sem)
            plsc.subcore_barrier()

            ooff = pl.multiple_of(sid * OPT, 8)
            pltpu.sync_copy(sdest.at[pl.ds(ooff, OPT)],
                            agg_h.at[pl.ds(pl.multiple_of(lo, 8) + ooff, OPT)])

            @pl.when(sid == _NS - 1)
            def _tail():
                pltpu.sync_copy(
                    sdest.at[pl.ds(_NS * OPT, R - _NS * OPT)],
                    agg_h.at[pl.ds(pl.multiple_of(lo, 8) + _NS * OPT,
                                   R - _NS * OPT)])
            plsc.subcore_barrier()

    return k(m, vw)


def kernel(h_pair, pair_vu_idx, pair_uw_idx, pair_vw_idx, geom_features,
           psi_W1, psi_b1, psi_W2, psi_b2, phi_W1, phi_b1, phi_W2, phi_b2):
    P, D = h_pair.shape
    vu = pair_vu_idx.astype(jnp.int32)
    uw = pair_uw_idx.astype(jnp.int32)
    vw = pair_vw_idx.astype(jnp.int32)

    wcat = jnp.concatenate(
        [psi_W1[0:D], psi_W1[D:2 * D], psi_W1[2 * D:3 * D], phi_W1[0:D]], axis=1)
    ha, hb, hc, hd = _precompute(h_pair, wcat)
    g = _gather_sum(ha, hb, hc, vu, uw, vw)
    m = _psi_tail(g, geom_features, psi_W1[3 * D:],
                  psi_b1.reshape(1, D), psi_W2, psi_b2.reshape(1, D))
    agg = _scatter_add(m, vw, P)
    out = _phi(h_pair, hd, agg, phi_W1[D:], phi_b1.reshape(1, D),
               phi_W2, phi_b2.reshape(1, D))
    return out
